# baseline (speedup 1.0000x reference)
import functools

import jax
import jax.numpy as jnp
from jax import lax
from jax.experimental import pallas as pl
from jax.experimental.pallas import tpu as pltpu

_INTERPRET = False

D = 256
IN_CH = 128
HEADS = 8
HD = D // HEADS
E = 8
MOE = 2 * D
HID = 1365
N = 2048
TOPK_THRESH = 0.2
BAL_COEF = 0.01
Z_COEF = 0.001

_RT = 256
_CT = 512


def _gelu(x):
    return x * 0.5 * (1.0 + lax.erf(x * (2.0 ** -0.5)))


def _embed_body(coords_ref, wt_ref, bt_ref, v_ref, wv_ref, bv_ref,
                topo_ref, val_ref):
    topo_ref[...] = _gelu(
        jnp.dot(coords_ref[...], wt_ref[...],
                preferred_element_type=jnp.float32) + bt_ref[...])
    val_ref[...] = _gelu(
        jnp.dot(v_ref[...], wv_ref[...],
                preferred_element_type=jnp.float32) + bv_ref[...])


def _embed(coords, W_topo, b_topo, V2, W_val, b_val):
    g = 2 * N // _CT
    return pl.pallas_call(
        _embed_body,
        grid=(g,),
        in_specs=[
            pl.BlockSpec((_CT, 3), lambda i: (i, 0)),
            pl.BlockSpec((3, D), lambda i: (0, 0)),
            pl.BlockSpec((1, D), lambda i: (0, 0)),
            pl.BlockSpec((_CT // 2, IN_CH), lambda i: (i, 0)),
            pl.BlockSpec((IN_CH, D), lambda i: (0, 0)),
            pl.BlockSpec((1, D), lambda i: (0, 0)),
        ],
        out_specs=[
            pl.BlockSpec((_CT, D), lambda i: (i, 0)),
            pl.BlockSpec((_CT // 2, D), lambda i: (i, 0)),
        ],
        out_shape=[
            jax.ShapeDtypeStruct((2 * N, D), jnp.float32),
            jax.ShapeDtypeStruct((N, D), jnp.float32),
        ],
        interpret=_INTERPRET,
    )(coords, W_topo, b_topo, V2, W_val, b_val)


def _qkv_body(q_in, k_in, v_in, w_ref, b_ref, q_o, k_o, v_o):
    w = w_ref[...]
    b = b_ref[...]
    q_o[...] = jnp.dot(q_in[...], w[:, :D],
                       preferred_element_type=jnp.float32) + b[:, :D]
    k_o[...] = jnp.dot(k_in[...], w[:, D:2 * D],
                       preferred_element_type=jnp.float32) + b[:, D:2 * D]
    v_o[...] = jnp.dot(v_in[...], w[:, 2 * D:],
                       preferred_element_type=jnp.float32) + b[:, 2 * D:]


def _qkv(queries, keys, values, W_in, b_in):
    g = N // _RT
    spec_r = pl.BlockSpec((_RT, D), lambda i: (i, 0))
    return pl.pallas_call(
        _qkv_body,
        grid=(g,),
        in_specs=[spec_r, spec_r, spec_r,
                  pl.BlockSpec((D, 3 * D), lambda i: (0, 0)),
                  pl.BlockSpec((1, 3 * D), lambda i: (0, 0))],
        out_specs=[spec_r, spec_r, spec_r],
        out_shape=[jax.ShapeDtypeStruct((N, D), jnp.float32)] * 3,
        interpret=_INTERPRET,
    )(queries, keys, values, W_in, b_in)


def _attn_body(q_ref, k_ref, v_ref, o_ref):
    s = lax.dot_general(q_ref[0], k_ref[0],
                        (((1,), (1,)), ((), ())),
                        preferred_element_type=jnp.float32)
    s = s * (HD ** -0.5)
    m = jnp.max(s, axis=1, keepdims=True)
    p = jnp.exp(s - m)
    l = jnp.sum(p, axis=1, keepdims=True)
    o = jnp.dot(p, v_ref[0], preferred_element_type=jnp.float32)
    o_ref[0] = o / l


def _attention(q, k, v):
    g = N // _RT
    return pl.pallas_call(
        _attn_body,
        grid=(HEADS, g),
        in_specs=[
            pl.BlockSpec((1, _RT, HD), lambda h, i: (h, i, 0)),
            pl.BlockSpec((1, N, HD), lambda h, i: (h, 0, 0)),
            pl.BlockSpec((1, N, HD), lambda h, i: (h, 0, 0)),
        ],
        out_specs=pl.BlockSpec((1, _RT, HD), lambda h, i: (h, i, 0)),
        out_shape=jax.ShapeDtypeStruct((HEADS, N, HD), jnp.float32),
        interpret=_INTERPRET,
    )(q, k, v)


def _router_body(attn_ref, qr_ref, wo_ref, bo_ref, g_ref, wg_ref,
                 x_ref, t_ref, c_ref, px_ref, dn_ref, zs_ref):
    i = pl.program_id(0)
    ao = jnp.dot(attn_ref[...], wo_ref[...],
                 preferred_element_type=jnp.float32) + bo_ref[...]
    x = jnp.concatenate([ao, qr_ref[...]], axis=1)
    x_ref[...] = x
    nrm = jnp.sqrt(jnp.sum(x * x, axis=1, keepdims=True) + 1e-12)
    t = x / nrm * (MOE ** 0.5) * g_ref[...]
    t_ref[...] = t
    logits = jnp.dot(t, wg_ref[...], preferred_element_type=jnp.float32)
    m = jnp.max(logits, axis=1, keepdims=True)
    ex = jnp.exp(logits - m)
    se = jnp.sum(ex, axis=1, keepdims=True)
    p = ex / se
    zl = m + jnp.log(se)
    iota = lax.broadcasted_iota(jnp.int32, (_RT, E), 1)
    m1 = jnp.max(p, axis=1, keepdims=True)
    i1 = jnp.min(jnp.where(p == m1, iota, E), axis=1, keepdims=True)
    pmask = jnp.where(iota == i1, -jnp.inf, p)
    m2 = jnp.max(pmask, axis=1, keepdims=True)
    i2 = jnp.min(jnp.where(pmask == m2, iota, E), axis=1, keepdims=True)
    w2 = m2 * (m2 >= TOPK_THRESH).astype(jnp.float32)
    oh1 = (iota == i1).astype(jnp.float32)
    oh2 = (iota == i2).astype(jnp.float32)
    c_ref[...] = oh1 * m1 + oh2 * w2
    px = jnp.sum(p, axis=0, keepdims=True)
    dn = jnp.sum(oh1, axis=0, keepdims=True)
    z = jnp.sum(zl * zl)
    zv = jnp.full((1, E), z, jnp.float32)

    @pl.when(i == 0)
    def _():
        px_ref[...] = px
        dn_ref[...] = dn
        zs_ref[...] = zv

    @pl.when(i > 0)
    def _():
        px_ref[...] += px
        dn_ref[...] += dn
        zs_ref[...] += zv


def _router(attn, queries, W_o, b_o, g_rms, W_gate):
    g = N // _RT
    spec_r = pl.BlockSpec((_RT, D), lambda i: (i, 0))
    spec_m = pl.BlockSpec((_RT, MOE), lambda i: (i, 0))
    spec_e = pl.BlockSpec((1, E), lambda i: (0, 0))
    return pl.pallas_call(
        _router_body,
        grid=(g,),
        in_specs=[spec_r, spec_r,
                  pl.BlockSpec((D, D), lambda i: (0, 0)),
                  pl.BlockSpec((1, D), lambda i: (0, 0)),
                  pl.BlockSpec((1, MOE), lambda i: (0, 0)),
                  pl.BlockSpec((MOE, E), lambda i: (0, 0))],
        out_specs=[spec_m, spec_m,
                   pl.BlockSpec((_RT, E), lambda i: (i, 0)),
                   spec_e, spec_e, spec_e],
        out_shape=[
            jax.ShapeDtypeStruct((N, MOE), jnp.float32),
            jax.ShapeDtypeStruct((N, MOE), jnp.float32),
            jax.ShapeDtypeStruct((N, E), jnp.float32),
            jax.ShapeDtypeStruct((1, E), jnp.float32),
            jax.ShapeDtypeStruct((1, E), jnp.float32),
            jax.ShapeDtypeStruct((1, E), jnp.float32),
        ],
        interpret=_INTERPRET,
    )(attn, queries, W_o, b_o, g_rms, W_gate)


def _moe_dense_body(t_ref, c_ref, lng_ref, lnb_ref, w1_ref, b1_ref,
                    w2_ref, b2_ref, o_ref):
    e = pl.program_id(1)
    t = t_ref[...]
    mu = jnp.mean(t, axis=1, keepdims=True)
    xc = t - mu
    var = jnp.mean(xc * xc, axis=1, keepdims=True)
    xn = xc * lax.rsqrt(var + 1e-5)
    h = xn * lng_ref[0] + lnb_ref[0]
    h1 = jnp.dot(h, w1_ref[0], preferred_element_type=jnp.float32) + b1_ref[0]
    h1 = jnp.where(h1 >= 0, h1, 0.01 * h1)
    h2 = jnp.dot(h1, w2_ref[0], preferred_element_type=jnp.float32) + b2_ref[0]
    lane = lax.broadcasted_iota(jnp.int32, (_RT, E), 1)
    cw = jnp.sum(jnp.where(lane == e, c_ref[...], 0.0), axis=1, keepdims=True)
    contrib = h2 * cw

    @pl.when(e == 0)
    def _():
        o_ref[...] = contrib

    @pl.when(e > 0)
    def _():
        o_ref[...] += contrib


def _moe_dense(t, combine, ln_g, ln_b, W1, b1, W2, b2):
    g = N // _RT
    return pl.pallas_call(
        _moe_dense_body,
        grid=(g, E),
        in_specs=[
            pl.BlockSpec((_RT, MOE), lambda i, e: (i, 0)),
            pl.BlockSpec((_RT, E), lambda i, e: (i, 0)),
            pl.BlockSpec((1, 1, MOE), lambda i, e: (e, 0, 0)),
            pl.BlockSpec((1, 1, MOE), lambda i, e: (e, 0, 0)),
            pl.BlockSpec((1, MOE, HID), lambda i, e: (e, 0, 0)),
            pl.BlockSpec((1, 1, HID), lambda i, e: (e, 0, 0)),
            pl.BlockSpec((1, HID, MOE), lambda i, e: (e, 0, 0)),
            pl.BlockSpec((1, 1, MOE), lambda i, e: (e, 0, 0)),
        ],
        out_specs=pl.BlockSpec((_RT, MOE), lambda i, e: (i, 0)),
        out_shape=jax.ShapeDtypeStruct((N, MOE), jnp.float32),
        interpret=_INTERPRET,
    )(t, combine, ln_g[:, None, :], ln_b[:, None, :], W1, b1[:, None, :],
      W2, b2[:, None, :])


def _final_body(moe_ref, x_ref, wl_ref, bl_ref, o_ref):
    y = moe_ref[...] + x_ref[...]
    o_ref[...] = _gelu(
        jnp.dot(y, wl_ref[...], preferred_element_type=jnp.float32)
        + bl_ref[...])


def _final(moe, x, W_lin, b_lin):
    g = N // _RT
    return pl.pallas_call(
        _final_body,
        grid=(g,),
        in_specs=[
            pl.BlockSpec((_RT, MOE), lambda i: (i, 0)),
            pl.BlockSpec((_RT, MOE), lambda i: (i, 0)),
            pl.BlockSpec((MOE, D), lambda i: (0, 0)),
            pl.BlockSpec((1, D), lambda i: (0, 0)),
        ],
        out_specs=pl.BlockSpec((_RT, D), lambda i: (i, 0)),
        out_shape=jax.ShapeDtypeStruct((N, D), jnp.float32),
        interpret=_INTERPRET,
    )(moe, x, W_lin, b_lin)


def kernel(K, V, Q, W_topo, b_topo, W_val, b_val, W_in, b_in, W_o, b_o,
           g_rms, W_gate, ln_g, ln_b, W1, b1, W2, b2, W_lin, b_lin):
    coords = jnp.concatenate([K[0], Q[0]], axis=0)
    topo, values = _embed(coords, W_topo, b_topo[None, :], V[0],
                          W_val, b_val[None, :])
    keys = topo[:N]
    queries = topo[N:]
    q, k, v = _qkv(queries, keys, values, W_in, b_in[None, :])

    def _heads(a):
        return a.reshape(N, HEADS, HD).transpose(1, 0, 2)

    attn3 = _attention(_heads(q), _heads(k), _heads(v))
    attn = attn3.transpose(1, 0, 2).reshape(N, D)
    x, t, combine, px, dn, zs = _router(attn, queries, W_o, b_o[None, :],
                                        g_rms[None, :], W_gate)
    moe = _moe_dense(t, combine, ln_g, ln_b, W1, b1, W2, b2)
    out = _final(moe, x, W_lin, b_lin[None, :])

    proxy_mean = px[0] / N
    dens_mean = dn[0] / N
    bal = jnp.mean(proxy_mean * dens_mean) * (E ** 2) * BAL_COEF
    zloss = (zs[0, 0] / N) * Z_COEF
    total = bal + zloss
    return out[None, :, :], total, bal, zloss
